# Initial kernel scaffold; baseline (speedup 1.0000x reference)
import jax
import jax.numpy as jnp
from jax.experimental import pallas as pl


def kernel(n_delay, n_res, edge_index, norm0_g, norm0_b, fc1_W, fc1_b, conv1_W, conv1_b, conv2_W, conv2_b, bn1_g, bn1_b, normres_g, normres_b, fcres_W, fcres_b, fc2_W, fc2_b, bn2_g, bn2_b, fc3_W, fc3_b):
    raise NotImplementedError("write your pallas kernel here")


# speedup vs baseline: 4.6782x; 4.6782x over previous
import functools

import jax
import jax.numpy as jnp
from jax import lax
from jax.experimental import pallas as pl
from jax.experimental.pallas import tpu as pltpu
from jax.experimental.pallas import tpu_sc as plsc

N = 50000
E = 1600000
DP = 32
NPAD = 50176
RT = NPAD // 16
LPT = 128
K = 56
G = 7
EPT = K * G * LPT
W = 32
EPAD = EPT * W

@functools.lru_cache(maxsize=None)
def _make_conv_sc():
    mesh = plsc.VectorSubcoreMesh(core_axis_name="c", subcore_axis_name="s")
    return functools.partial(
        pl.kernel,
        out_type=jax.ShapeDtypeStruct((2, NPAD, DP), jnp.float32),
        mesh=mesh,
        scratch_types=[
            pltpu.VMEM((K, LPT), jnp.int32),
            pltpu.VMEM((K, LPT), jnp.int32),
            pltpu.VMEM((LPT, DP), jnp.float32),
            pltpu.VMEM_SHARED((NPAD, DP), jnp.float32),
            pltpu.SemaphoreType.DMA,
        ],
        compiler_params=pltpu.CompilerParams(use_tc_tiling_on_sc=False),
    )(_conv_sc_body)


def _conv_sc_body(h_hbm, src_hbm, dst_hbm, zrow_hbm, out_hbm,
                  sidx, didx, rows, agg, sem):
    cid = lax.axis_index("c")
    sid = lax.axis_index("s")
    w = sid * 2 + cid

    pltpu.sync_copy(zrow_hbm, agg.at[pl.ds(sid * RT, RT)])
    plsc.subcore_barrier()

    def group(g, _):
        rowbase = w * (G * K) + g * K
        pltpu.sync_copy(src_hbm.at[pl.ds(rowbase, K)], sidx)
        pltpu.sync_copy(dst_hbm.at[pl.ds(rowbase, K)], didx)

        def inner(j, _):
            pltpu.async_copy(h_hbm.at[sidx.at[j]], rows, sem).wait()
            pltpu.sync_copy(rows, agg.at[didx.at[j]], add=True)
            return 0

        lax.fori_loop(0, K, inner, 0)
        return 0

    lax.fori_loop(0, G, group, 0)
    plsc.subcore_barrier()

    pltpu.sync_copy(agg.at[pl.ds(sid * RT, RT)],
                    out_hbm.at[cid, pl.ds(sid * RT, RT)])


def _bn(x, g, b):
    m = jnp.mean(x, axis=0)
    v = jnp.var(x, axis=0)
    return (x - m) * jax.lax.rsqrt(v + 1e-5) * g + b


def kernel(n_delay, n_res, edge_index, norm0_g, norm0_b, fc1_W, fc1_b,
           conv1_W, conv1_b, conv2_W, conv2_b, bn1_g, bn1_b, normres_g,
           normres_b, fcres_W, fcres_b, fc2_W, fc2_b, bn2_g, bn2_b,
           fc3_W, fc3_b):
    src = edge_index[0]
    dst = edge_index[1]

    pad = jnp.full((EPAD - E,), N, jnp.int32)
    src2d = jnp.concatenate([src, pad]).reshape(-1, LPT)
    dst2d = jnp.concatenate([dst, pad]).reshape(-1, LPT)
    zrow = jnp.zeros((RT, DP), jnp.float32)

    out_deg = jnp.clip(jnp.zeros((N,), jnp.float32).at[src].add(1.0), 1.0)
    s_out = lax.rsqrt(out_deg)

    ip = jnp.concatenate([n_delay, n_res], axis=1)
    h1 = jax.nn.relu(_bn(ip, norm0_g, norm0_b) @ fc1_W + fc1_b)

    def conv(x, Wm, b, with_ones):
        hw = (x @ Wm) * s_out[:, None]
        h_pad = jnp.zeros((NPAD, DP), jnp.float32)
        h_pad = h_pad.at[:N, :24].set(hw)
        if with_ones:
            h_pad = h_pad.at[:N, 24].set(1.0)
        p = _make_conv_sc()(h_pad, src2d, dst2d, zrow)
        s = p[0] + p[1]
        return s[:N, :24] + b, s[:N, 24]

    a1, in_deg = conv(h1, conv1_W, jnp.zeros((24,), jnp.float32), True)
    s_in = lax.rsqrt(jnp.clip(in_deg, 1.0))
    g1 = a1 * s_in[:, None] + conv1_b
    a2, _ = conv(g1, conv2_W, jnp.zeros((24,), jnp.float32), False)
    g2 = a2 * s_in[:, None] + conv2_b

    hres = _bn(n_res, normres_g, normres_b) @ fcres_W + fcres_b
    ht = jax.nn.relu(_bn(g2, bn1_g, bn1_b) @ fc2_W + fc2_b)
    h2 = jnp.concatenate([ht, hres], axis=1)
    action = jax.nn.softmax(_bn(h2, bn2_g, bn2_b) @ fc3_W + fc3_b, axis=1)
    return action


# speedup vs baseline: 10.3471x; 2.2118x over previous
import functools

import jax
import jax.numpy as jnp
from jax import lax
from jax.experimental import pallas as pl
from jax.experimental.pallas import tpu as pltpu
from jax.experimental.pallas import tpu_sc as plsc

N = 50000
E = 1600000
DP = 32
NPAD = 50176
RT = NPAD // 16
LPT = 128
K = 56
G = 7
EPT = K * G * LPT
W = 32
EPAD = EPT * W

@functools.lru_cache(maxsize=None)
def _make_conv_sc():
    mesh = plsc.VectorSubcoreMesh(core_axis_name="c", subcore_axis_name="s")
    return functools.partial(
        pl.kernel,
        out_type=jax.ShapeDtypeStruct((2, NPAD, DP), jnp.float32),
        mesh=mesh,
        scratch_types=[
            pltpu.VMEM((K, LPT), jnp.int32),
            pltpu.VMEM((K, LPT), jnp.int32),
            pltpu.VMEM((LPT, DP), jnp.float32),
            pltpu.VMEM_SHARED((NPAD, DP), jnp.float32),
            pltpu.SemaphoreType.DMA,
        ],
        compiler_params=pltpu.CompilerParams(use_tc_tiling_on_sc=False),
    )(_conv_sc_body)


def _conv_sc_body(h_hbm, src_hbm, dst_hbm, zrow_hbm, out_hbm,
                  sidx, didx, rows, agg, sem):
    cid = lax.axis_index("c")
    sid = lax.axis_index("s")
    w = sid * 2 + cid

    pltpu.sync_copy(zrow_hbm, agg.at[pl.ds(sid * RT, RT)])
    plsc.subcore_barrier()

    def group(g, _):
        rowbase = w * (G * K) + g * K
        pltpu.sync_copy(src_hbm.at[pl.ds(rowbase, K)], sidx)
        pltpu.sync_copy(dst_hbm.at[pl.ds(rowbase, K)], didx)

        def inner(j, _):
            pltpu.async_copy(h_hbm.at[sidx.at[j]], rows, sem).wait()
            pltpu.sync_copy(rows, agg.at[didx.at[j]], add=True)
            return 0

        lax.fori_loop(0, K, inner, 0)
        return 0

    lax.fori_loop(0, G, group, 0)
    plsc.subcore_barrier()

    pltpu.sync_copy(agg.at[pl.ds(sid * RT, RT)],
                    out_hbm.at[cid, pl.ds(sid * RT, RT)])


@functools.lru_cache(maxsize=None)
def _make_deg_sc():
    mesh = plsc.VectorSubcoreMesh(core_axis_name="c", subcore_axis_name="s")
    return functools.partial(
        pl.kernel,
        out_type=jax.ShapeDtypeStruct((2, NPAD), jnp.float32),
        mesh=mesh,
        scratch_types=[
            pltpu.VMEM((K, LPT), jnp.int32),
            pltpu.VMEM((LPT,), jnp.float32),
            pltpu.VMEM_SHARED((NPAD,), jnp.float32),
        ],
        compiler_params=pltpu.CompilerParams(use_tc_tiling_on_sc=False),
    )(_deg_sc_body)


def _deg_sc_body(src_hbm, zdeg_hbm, out_hbm, sidx, ones_v, counts):
    cid = lax.axis_index("c")
    sid = lax.axis_index("s")
    w = sid * 2 + cid
    for i in range(LPT // 16):
        ones_v[pl.ds(i * 16, 16)] = jnp.ones((16,), jnp.float32)
    pltpu.sync_copy(zdeg_hbm, counts.at[pl.ds(sid * RT, RT)])
    plsc.subcore_barrier()

    def group(g, _):
        pltpu.sync_copy(src_hbm.at[pl.ds(w * (G * K) + g * K, K)], sidx)

        def inner(j, _):
            pltpu.sync_copy(ones_v, counts.at[sidx.at[j]], add=True)
            return 0

        lax.fori_loop(0, K, inner, 0)
        return 0

    lax.fori_loop(0, G, group, 0)
    plsc.subcore_barrier()
    pltpu.sync_copy(counts.at[pl.ds(sid * RT, RT)],
                    out_hbm.at[cid, pl.ds(sid * RT, RT)])


def _bn(x, g, b):
    m = jnp.mean(x, axis=0)
    v = jnp.var(x, axis=0)
    return (x - m) * jax.lax.rsqrt(v + 1e-5) * g + b


def kernel(n_delay, n_res, edge_index, norm0_g, norm0_b, fc1_W, fc1_b,
           conv1_W, conv1_b, conv2_W, conv2_b, bn1_g, bn1_b, normres_g,
           normres_b, fcres_W, fcres_b, fc2_W, fc2_b, bn2_g, bn2_b,
           fc3_W, fc3_b):
    src = edge_index[0]
    dst = edge_index[1]

    pad = jnp.full((EPAD - E,), N, jnp.int32)
    src2d = jnp.concatenate([src, pad]).reshape(-1, LPT)
    dst2d = jnp.concatenate([dst, pad]).reshape(-1, LPT)
    zrow = jnp.zeros((RT, DP), jnp.float32)
    zdeg = jnp.zeros((RT,), jnp.float32)

    dp = _make_deg_sc()(src2d, zdeg)
    s_out = lax.rsqrt(jnp.clip(dp[0, :N] + dp[1, :N], 1.0))

    ip = jnp.concatenate([n_delay, n_res], axis=1)
    h1 = jax.nn.relu(_bn(ip, norm0_g, norm0_b) @ fc1_W + fc1_b)

    def conv(x, Wm, b, with_ones):
        hw = (x @ Wm) * s_out[:, None]
        h_pad = jnp.zeros((NPAD, DP), jnp.float32)
        h_pad = h_pad.at[:N, :24].set(hw)
        if with_ones:
            h_pad = h_pad.at[:N, 24].set(1.0)
        p = _make_conv_sc()(h_pad, src2d, dst2d, zrow)
        s = p[0] + p[1]
        return s[:N, :24] + b, s[:N, 24]

    a1, in_deg = conv(h1, conv1_W, jnp.zeros((24,), jnp.float32), True)
    s_in = lax.rsqrt(jnp.clip(in_deg, 1.0))
    g1 = a1 * s_in[:, None] + conv1_b
    a2, _ = conv(g1, conv2_W, jnp.zeros((24,), jnp.float32), False)
    g2 = a2 * s_in[:, None] + conv2_b

    hres = _bn(n_res, normres_g, normres_b) @ fcres_W + fcres_b
    ht = jax.nn.relu(_bn(g2, bn1_g, bn1_b) @ fc2_W + fc2_b)
    h2 = jnp.concatenate([ht, hres], axis=1)
    action = jax.nn.softmax(_bn(h2, bn2_g, bn2_b) @ fc3_W + fc3_b, axis=1)
    return action


# speedup vs baseline: 14.9535x; 1.4452x over previous
import functools

import jax
import jax.numpy as jnp
from jax import lax
from jax.experimental import pallas as pl
from jax.experimental.pallas import tpu as pltpu
from jax.experimental.pallas import tpu_sc as plsc

N = 50000
E = 1600000
DP = 32
NPAD = 50176
RT = NPAD // 16
LPT = 128
NB2 = 2
NSL = 3
T = 196
EPT = T * NB2 * LPT
W = 32
EPAD = EPT * W

@functools.lru_cache(maxsize=None)
def _make_conv_sc():
    mesh = plsc.VectorSubcoreMesh(core_axis_name="c", subcore_axis_name="s")
    return functools.partial(
        pl.kernel,
        out_type=jax.ShapeDtypeStruct((2, NPAD, DP), jnp.float32),
        mesh=mesh,
        scratch_types=[
            pltpu.VMEM((NSL, NB2, LPT), jnp.int32),
            pltpu.VMEM((NSL, NB2, LPT), jnp.int32),
            pltpu.VMEM((NSL, NB2, LPT, DP), jnp.float32),
            pltpu.VMEM_SHARED((NPAD, DP), jnp.float32),
            pltpu.SemaphoreType.DMA,
            pltpu.SemaphoreType.DMA,
            pltpu.SemaphoreType.DMA,
        ],
        compiler_params=pltpu.CompilerParams(use_tc_tiling_on_sc=False),
    )(_conv_sc_body)


def _conv_sc_body(h_hbm, src_hbm, dst_hbm, zrow_hbm, out_hbm,
                  sidx, didx, rows, agg, gsem, ssem, isem):
    cid = lax.axis_index("c")
    sid = lax.axis_index("s")
    w = sid * 2 + cid
    rbase = w * (T * NB2)

    pltpu.sync_copy(zrow_hbm, agg.at[pl.ds(sid * RT, RT)])
    plsc.subcore_barrier()

    pltpu.async_copy(src_hbm.at[pl.ds(rbase, NB2)], sidx.at[0], isem)
    pltpu.async_copy(dst_hbm.at[pl.ds(rbase, NB2)], didx.at[0], isem)

    def _drain_scatters(s):
        for i in range(NB2):
            pltpu.make_async_copy(rows.at[s, i], agg.at[didx.at[s, i]],
                                  ssem).wait()

    def body(b, _):
        slot = lax.rem(b, NSL)

        @pl.when(b >= 2)
        def _():
            _drain_scatters(lax.rem(b + NSL - 2, NSL))

        @pl.when(b + 1 < T)
        def _():
            ns = lax.rem(b + 1, NSL)
            nb = rbase + (b + 1) * NB2
            pltpu.async_copy(src_hbm.at[pl.ds(nb, NB2)], sidx.at[ns], isem)
            pltpu.async_copy(dst_hbm.at[pl.ds(nb, NB2)], didx.at[ns], isem)

        pltpu.make_async_copy(src_hbm.at[pl.ds(rbase, NB2)],
                              sidx.at[slot], isem).wait()
        pltpu.make_async_copy(dst_hbm.at[pl.ds(rbase, NB2)],
                              didx.at[slot], isem).wait()

        for i in range(NB2):
            pltpu.async_copy(h_hbm.at[sidx.at[slot, i]], rows.at[slot, i],
                             gsem)

        @pl.when(b >= 1)
        def _():
            os_ = lax.rem(b + NSL - 1, NSL)
            for i in range(NB2):
                pltpu.make_async_copy(h_hbm.at[sidx.at[os_, i]],
                                      rows.at[os_, i], gsem).wait()
            for i in range(NB2):
                pltpu.async_copy(rows.at[os_, i], agg.at[didx.at[os_, i]],
                                 ssem, add=True)
        return 0

    lax.fori_loop(0, T, body, 0)

    ls = (T - 1) % NSL
    for i in range(NB2):
        pltpu.make_async_copy(h_hbm.at[sidx.at[ls, i]], rows.at[ls, i],
                              gsem).wait()
    for i in range(NB2):
        pltpu.async_copy(rows.at[ls, i], agg.at[didx.at[ls, i]], ssem,
                         add=True)
    _drain_scatters((T - 2) % NSL)
    _drain_scatters(ls)

    plsc.subcore_barrier()
    pltpu.sync_copy(agg.at[pl.ds(sid * RT, RT)],
                    out_hbm.at[cid, pl.ds(sid * RT, RT)])


@functools.lru_cache(maxsize=None)
def _make_deg_sc():
    mesh = plsc.VectorSubcoreMesh(core_axis_name="c", subcore_axis_name="s")
    return functools.partial(
        pl.kernel,
        out_type=jax.ShapeDtypeStruct((2, NPAD), jnp.float32),
        mesh=mesh,
        scratch_types=[
            pltpu.VMEM((NSL, NB2, LPT), jnp.int32),
            pltpu.VMEM((LPT,), jnp.float32),
            pltpu.VMEM_SHARED((NPAD,), jnp.float32),
            pltpu.SemaphoreType.DMA,
            pltpu.SemaphoreType.DMA,
        ],
        compiler_params=pltpu.CompilerParams(use_tc_tiling_on_sc=False),
    )(_deg_sc_body)


def _deg_sc_body(src_hbm, zdeg_hbm, out_hbm, sidx, ones_v, counts,
                 ssem, isem):
    cid = lax.axis_index("c")
    sid = lax.axis_index("s")
    w = sid * 2 + cid
    rbase = w * (T * NB2)
    for i in range(LPT // 16):
        ones_v[pl.ds(i * 16, 16)] = jnp.ones((16,), jnp.float32)
    pltpu.sync_copy(zdeg_hbm, counts.at[pl.ds(sid * RT, RT)])
    plsc.subcore_barrier()

    pltpu.async_copy(src_hbm.at[pl.ds(rbase, NB2)], sidx.at[0], isem)

    def _drain(s):
        for i in range(NB2):
            pltpu.make_async_copy(ones_v, counts.at[sidx.at[s, i]],
                                  ssem).wait()

    def body(b, _):
        slot = lax.rem(b, NSL)

        @pl.when(b >= 2)
        def _():
            _drain(lax.rem(b + NSL - 2, NSL))

        @pl.when(b + 1 < T)
        def _():
            ns = lax.rem(b + 1, NSL)
            pltpu.async_copy(src_hbm.at[pl.ds(rbase + (b + 1) * NB2, NB2)],
                             sidx.at[ns], isem)

        pltpu.make_async_copy(src_hbm.at[pl.ds(rbase, NB2)],
                              sidx.at[slot], isem).wait()

        for i in range(NB2):
            pltpu.async_copy(ones_v, counts.at[sidx.at[slot, i]], ssem,
                             add=True)
        return 0

    lax.fori_loop(0, T, body, 0)
    for d in (2, 1):
        _drain((T - d) % NSL)
    plsc.subcore_barrier()
    pltpu.sync_copy(counts.at[pl.ds(sid * RT, RT)],
                    out_hbm.at[cid, pl.ds(sid * RT, RT)])


def _bn(x, g, b):
    m = jnp.mean(x, axis=0)
    v = jnp.var(x, axis=0)
    return (x - m) * jax.lax.rsqrt(v + 1e-5) * g + b


def kernel(n_delay, n_res, edge_index, norm0_g, norm0_b, fc1_W, fc1_b,
           conv1_W, conv1_b, conv2_W, conv2_b, bn1_g, bn1_b, normres_g,
           normres_b, fcres_W, fcres_b, fc2_W, fc2_b, bn2_g, bn2_b,
           fc3_W, fc3_b):
    src = edge_index[0]
    dst = edge_index[1]

    pad = jnp.full((EPAD - E,), N, jnp.int32)
    src2d = jnp.concatenate([src, pad]).reshape(-1, LPT)
    dst2d = jnp.concatenate([dst, pad]).reshape(-1, LPT)
    zrow = jnp.zeros((RT, DP), jnp.float32)
    zdeg = jnp.zeros((RT,), jnp.float32)

    dp = _make_deg_sc()(src2d, zdeg)
    s_out = lax.rsqrt(jnp.clip(dp[0, :N] + dp[1, :N], 1.0))

    ip = jnp.concatenate([n_delay, n_res], axis=1)
    h1 = jax.nn.relu(_bn(ip, norm0_g, norm0_b) @ fc1_W + fc1_b)

    def conv(x, Wm, b, with_ones):
        hw = (x @ Wm) * s_out[:, None]
        h_pad = jnp.zeros((NPAD, DP), jnp.float32)
        h_pad = h_pad.at[:N, :24].set(hw)
        if with_ones:
            h_pad = h_pad.at[:N, 24].set(1.0)
        p = _make_conv_sc()(h_pad, src2d, dst2d, zrow)
        s = p[0] + p[1]
        return s[:N, :24] + b, s[:N, 24]

    a1, in_deg = conv(h1, conv1_W, jnp.zeros((24,), jnp.float32), True)
    s_in = lax.rsqrt(jnp.clip(in_deg, 1.0))
    g1 = a1 * s_in[:, None] + conv1_b
    a2, _ = conv(g1, conv2_W, jnp.zeros((24,), jnp.float32), False)
    g2 = a2 * s_in[:, None] + conv2_b

    hres = _bn(n_res, normres_g, normres_b) @ fcres_W + fcres_b
    ht = jax.nn.relu(_bn(g2, bn1_g, bn1_b) @ fc2_W + fc2_b)
    h2 = jnp.concatenate([ht, hres], axis=1)
    action = jax.nn.softmax(_bn(h2, bn2_g, bn2_b) @ fc3_W + fc3_b, axis=1)
    return action


# speedup vs baseline: 19.8429x; 1.3270x over previous
import functools

import jax
import jax.numpy as jnp
from jax import lax
from jax.experimental import pallas as pl
from jax.experimental.pallas import tpu as pltpu
from jax.experimental.pallas import tpu_sc as plsc

N = 50000
E = 1600000
ER = E // 128
DP = 32
NPAD = 50176
RT = NPAD // 16
LPT = 128
NB2 = 2
NSL = 3
T = 196
TLAST = 174
EPS = 1e-5

BR = 2048
GRID = (N + BR - 1) // BR


@functools.lru_cache(maxsize=None)
def _make_conv_sc():
    mesh = plsc.VectorSubcoreMesh(core_axis_name="c", subcore_axis_name="s")
    return functools.partial(
        pl.kernel,
        out_type=jax.ShapeDtypeStruct((2, NPAD, DP), jnp.float32),
        mesh=mesh,
        scratch_types=[
            pltpu.VMEM((NSL, NB2, LPT), jnp.int32),
            pltpu.VMEM((NSL, NB2, LPT), jnp.int32),
            pltpu.VMEM((NSL, NB2, LPT, DP), jnp.float32),
            pltpu.VMEM_SHARED((NPAD, DP), jnp.float32),
            pltpu.SemaphoreType.DMA,
            pltpu.SemaphoreType.DMA,
            pltpu.SemaphoreType.DMA,
        ],
        compiler_params=pltpu.CompilerParams(use_tc_tiling_on_sc=False),
    )(_conv_sc_body)


def _conv_sc_body(h_hbm, edge_hbm, zrow_hbm, out_hbm,
                  sidx, didx, rows, agg, gsem, ssem, isem):
    cid = lax.axis_index("c")
    sid = lax.axis_index("s")
    w = sid * 2 + cid
    rbase = w * (T * NB2)
    nb = jnp.where(w == 31, TLAST, T)

    pltpu.sync_copy(zrow_hbm, agg.at[pl.ds(sid * RT, RT)])
    plsc.subcore_barrier()

    pltpu.async_copy(edge_hbm.at[0, pl.ds(rbase, NB2)], sidx.at[0], isem)
    pltpu.async_copy(edge_hbm.at[1, pl.ds(rbase, NB2)], didx.at[0], isem)

    def _drain_scatters(s):
        for i in range(NB2):
            pltpu.make_async_copy(rows.at[s, i], agg.at[didx.at[s, i]],
                                  ssem).wait()

    def _drain_gathers(s):
        for i in range(NB2):
            pltpu.make_async_copy(h_hbm.at[sidx.at[s, i]], rows.at[s, i],
                                  gsem).wait()

    def body(b, _):
        slot = lax.rem(b, NSL)

        @pl.when(b >= 2)
        def _():
            _drain_scatters(lax.rem(b + NSL - 2, NSL))

        @pl.when(b + 1 < nb)
        def _():
            ns = lax.rem(b + 1, NSL)
            r = rbase + (b + 1) * NB2
            pltpu.async_copy(edge_hbm.at[0, pl.ds(r, NB2)], sidx.at[ns], isem)
            pltpu.async_copy(edge_hbm.at[1, pl.ds(r, NB2)], didx.at[ns], isem)

        pltpu.make_async_copy(edge_hbm.at[0, pl.ds(rbase, NB2)],
                              sidx.at[slot], isem).wait()
        pltpu.make_async_copy(edge_hbm.at[1, pl.ds(rbase, NB2)],
                              didx.at[slot], isem).wait()

        for i in range(NB2):
            pltpu.async_copy(h_hbm.at[sidx.at[slot, i]], rows.at[slot, i],
                             gsem)

        @pl.when(b >= 1)
        def _():
            os_ = lax.rem(b + NSL - 1, NSL)
            _drain_gathers(os_)
            for i in range(NB2):
                pltpu.async_copy(rows.at[os_, i], agg.at[didx.at[os_, i]],
                                 ssem, add=True)
        return 0

    lax.fori_loop(0, nb, body, 0)

    ls = lax.rem(nb + NSL - 1, NSL)
    _drain_gathers(ls)
    for i in range(NB2):
        pltpu.async_copy(rows.at[ls, i], agg.at[didx.at[ls, i]], ssem,
                         add=True)
    _drain_scatters(lax.rem(nb + NSL - 2, NSL))
    _drain_scatters(ls)

    plsc.subcore_barrier()
    pltpu.sync_copy(agg.at[pl.ds(sid * RT, RT)],
                    out_hbm.at[cid, pl.ds(sid * RT, RT)])


@functools.lru_cache(maxsize=None)
def _make_deg_sc():
    mesh = plsc.VectorSubcoreMesh(core_axis_name="c", subcore_axis_name="s")
    return functools.partial(
        pl.kernel,
        out_type=jax.ShapeDtypeStruct((2, NPAD), jnp.float32),
        mesh=mesh,
        scratch_types=[
            pltpu.VMEM((NSL, NB2, LPT), jnp.int32),
            pltpu.VMEM((LPT,), jnp.float32),
            pltpu.VMEM_SHARED((NPAD,), jnp.float32),
            pltpu.SemaphoreType.DMA,
            pltpu.SemaphoreType.DMA,
        ],
        compiler_params=pltpu.CompilerParams(use_tc_tiling_on_sc=False),
    )(_deg_sc_body)


def _deg_sc_body(edge_hbm, zdeg_hbm, out_hbm, sidx, ones_v, counts,
                 ssem, isem):
    cid = lax.axis_index("c")
    sid = lax.axis_index("s")
    w = sid * 2 + cid
    rbase = w * (T * NB2)
    nb = jnp.where(w == 31, TLAST, T)
    for i in range(LPT // 16):
        ones_v[pl.ds(i * 16, 16)] = jnp.ones((16,), jnp.float32)
    pltpu.sync_copy(zdeg_hbm, counts.at[pl.ds(sid * RT, RT)])
    plsc.subcore_barrier()

    pltpu.async_copy(edge_hbm.at[0, pl.ds(rbase, NB2)], sidx.at[0], isem)

    def _drain(s):
        for i in range(NB2):
            pltpu.make_async_copy(ones_v, counts.at[sidx.at[s, i]],
                                  ssem).wait()

    def body(b, _):
        slot = lax.rem(b, NSL)

        @pl.when(b >= 2)
        def _():
            _drain(lax.rem(b + NSL - 2, NSL))

        @pl.when(b + 1 < nb)
        def _():
            ns = lax.rem(b + 1, NSL)
            pltpu.async_copy(edge_hbm.at[0, pl.ds(rbase + (b + 1) * NB2, NB2)],
                             sidx.at[ns], isem)

        pltpu.make_async_copy(edge_hbm.at[0, pl.ds(rbase, NB2)],
                              sidx.at[slot], isem).wait()

        for i in range(NB2):
            pltpu.async_copy(ones_v, counts.at[sidx.at[slot, i]], ssem,
                             add=True)
        return 0

    lax.fori_loop(0, nb, body, 0)
    _drain(lax.rem(nb + NSL - 2, NSL))
    _drain(lax.rem(nb + NSL - 1, NSL))
    plsc.subcore_barrier()
    pltpu.sync_copy(counts.at[pl.ds(sid * RT, RT)],
                    out_hbm.at[cid, pl.ds(sid * RT, RT)])


def _row_mask(i, br):
    rid = i * BR + lax.broadcasted_iota(jnp.int32, (BR, 1), 0)
    return rid < N


def _stats0_body(nd_ref, nr_ref, out_ref):
    i = pl.program_id(0)

    @pl.when(i == 0)
    def _():
        out_ref[...] = jnp.zeros_like(out_ref)

    m = _row_mask(i, BR)
    x = jnp.concatenate([nd_ref[...], nr_ref[...]], axis=1)
    x = jnp.where(m, x, 0.0)
    out_ref[0:1, :] += jnp.sum(x, axis=0)[None, :]
    out_ref[1:2, :] += jnp.sum(x * x, axis=0)[None, :]


def _front_body(nd_ref, nr_ref, w1d_ref, w1r_ref, b1_ref, wp_ref, dg_ref,
                out_ref):
    h1 = jax.nn.relu(
        jnp.dot(nd_ref[...], w1d_ref[...], preferred_element_type=jnp.float32)
        + jnp.dot(nr_ref[...], w1r_ref[...],
                  preferred_element_type=jnp.float32)
        + b1_ref[0:1, :])
    hw = jnp.dot(h1, wp_ref[...], preferred_element_type=jnp.float32)
    s_out = lax.rsqrt(jnp.maximum(dg_ref[...], 1.0))[:, None]
    one24 = jnp.where(
        lax.broadcasted_iota(jnp.int32, (1, DP), 1) == 24, 1.0, 0.0)
    out_ref[...] = hw * s_out + one24


def _mid_body(p0_ref, p1_ref, b1p_ref, w2_ref, dg_ref, out_ref, sin_ref):
    s = p0_ref[...] + p1_ref[...]
    ind = jnp.maximum(jnp.sum(s * jnp.where(
        lax.broadcasted_iota(jnp.int32, (1, DP), 1) == 24, 1.0, 0.0),
        axis=1, keepdims=True), 1.0)
    sin = lax.rsqrt(ind)
    g1 = s * sin + b1p_ref[0:1, :]
    hw = jnp.dot(g1, w2_ref[...], preferred_element_type=jnp.float32)
    s_out = lax.rsqrt(jnp.maximum(dg_ref[...], 1.0))[:, None]
    out_ref[...] = hw * s_out
    sin_ref[...] = sin[:, 0]


def _stats1_body(p0_ref, p1_ref, sin_ref, b2p_ref, out_ref):
    i = pl.program_id(0)

    @pl.when(i == 0)
    def _():
        out_ref[...] = jnp.zeros_like(out_ref)

    g2 = ((p0_ref[...] + p1_ref[...]) * sin_ref[...][:, None]
          + b2p_ref[0:1, :])
    g2 = jnp.where(_row_mask(i, BR), g2, 0.0)
    out_ref[0:1, :] += jnp.pad(jnp.sum(g2, axis=0), (0, 128 - DP))[None, :]
    out_ref[1:2, :] += jnp.pad(jnp.sum(g2 * g2, axis=0),
                               (0, 128 - DP))[None, :]


def _mix_body(p0_ref, p1_ref, sin_ref, nr_ref, b2p_ref, sc1_ref, sh1_ref,
              w2f_ref, b2f_ref, scr_ref, shr_ref, wr_ref, br_ref,
              out_ref, st_ref):
    i = pl.program_id(0)

    @pl.when(i == 0)
    def _():
        st_ref[...] = jnp.zeros_like(st_ref)

    g2 = ((p0_ref[...] + p1_ref[...]) * sin_ref[...][:, None]
          + b2p_ref[0:1, :])
    g2n = g2 * sc1_ref[0:1, :] + sh1_ref[0:1, :]
    ht = jax.nn.relu(
        jnp.dot(g2n, w2f_ref[...], preferred_element_type=jnp.float32)
        + b2f_ref[0:1, :])
    nrn = nr_ref[...] * scr_ref[0:1, :] + shr_ref[0:1, :]
    hres = (jnp.dot(nrn, wr_ref[...], preferred_element_type=jnp.float32)
            + br_ref[0:1, :])
    h2 = jnp.concatenate([ht, hres], axis=1)
    out_ref[...] = h2
    h2m = jnp.where(_row_mask(i, BR), h2, 0.0)
    st_ref[0:1, :] += jnp.pad(jnp.sum(h2m, axis=0), (0, 80))[None, :]
    st_ref[1:2, :] += jnp.pad(jnp.sum(h2m * h2m, axis=0), (0, 80))[None, :]


def _head_body(h2_ref, sc2_ref, sh2_ref, w3_ref, b3_ref, out_ref):
    z = (jnp.dot(h2_ref[...] * sc2_ref[0:1, :] + sh2_ref[0:1, :],
                 w3_ref[...], preferred_element_type=jnp.float32)
         + b3_ref[0:1, :])
    z = z - jnp.max(z, axis=1, keepdims=True)
    e = jnp.exp(z)
    out_ref[...] = e / jnp.sum(e, axis=1, keepdims=True)


def _full(shape):
    return pl.BlockSpec(shape, lambda i: tuple(0 for _ in shape))


def _rows(width):
    return pl.BlockSpec((BR, width), lambda i: (i, 0))


def _rows1():
    return pl.BlockSpec((BR,), lambda i: (i,))


def kernel(n_delay, n_res, edge_index, norm0_g, norm0_b, fc1_W, fc1_b,
           conv1_W, conv1_b, conv2_W, conv2_b, bn1_g, bn1_b, normres_g,
           normres_b, fcres_W, fcres_b, fc2_W, fc2_b, bn2_g, bn2_b,
           fc3_W, fc3_b):
    f32 = jnp.float32
    edge3d = edge_index.reshape(2, ER, LPT)
    zrow = jnp.zeros((RT, DP), f32)
    zdeg = jnp.zeros((RT,), f32)

    dgp = _make_deg_sc()(edge3d, zdeg)
    deg = dgp[0, :N] + dgp[1, :N]

    st0 = pl.pallas_call(
        _stats0_body, grid=GRID,
        in_specs=[_rows(112), _rows(16)],
        out_specs=_full((8, 128)),
        out_shape=jax.ShapeDtypeStruct((8, 128), f32),
    )(n_delay, n_res)
    m0 = st0[0] / N
    v0 = st0[1] / N - m0 * m0
    sc0 = lax.rsqrt(v0 + EPS) * norm0_g
    sh0 = norm0_b - m0 * sc0

    w1 = sc0[:, None] * fc1_W
    b1 = (sh0 @ fc1_W + fc1_b)[None, :]
    wp = jnp.pad(conv1_W, ((0, 0), (0, DP - 24)))
    b1p = jnp.pad(conv1_b, (0, DP - 24))[None, :]
    w2 = jnp.pad(conv2_W, ((0, DP - 24), (0, DP - 24)))
    b2p = jnp.pad(conv2_b, (0, DP - 24))[None, :]

    h1t = pl.pallas_call(
        _front_body, grid=GRID,
        in_specs=[_rows(112), _rows(16), _full((112, 40)), _full((16, 40)),
                  _full((1, 40)), _full((40, DP)), _rows1()],
        out_specs=_rows(DP),
        out_shape=jax.ShapeDtypeStruct((N, DP), f32),
    )(n_delay, n_res, w1[:112], w1[112:], b1, wp, deg)

    p1 = _make_conv_sc()(h1t, edge3d, zrow)

    h2t, sin = pl.pallas_call(
        _mid_body, grid=GRID,
        in_specs=[_rows(DP), _rows(DP), _full((1, DP)), _full((DP, DP)),
                  _rows1()],
        out_specs=(_rows(DP), _rows1()),
        out_shape=(jax.ShapeDtypeStruct((N, DP), f32),
                   jax.ShapeDtypeStruct((N,), f32)),
    )(p1[0], p1[1], b1p, w2, deg)

    p2 = _make_conv_sc()(h2t, edge3d, zrow)

    st1 = pl.pallas_call(
        _stats1_body, grid=GRID,
        in_specs=[_rows(DP), _rows(DP), _rows1(), _full((1, DP))],
        out_specs=_full((8, 128)),
        out_shape=jax.ShapeDtypeStruct((8, 128), f32),
    )(p2[0], p2[1], sin, b2p)
    m1 = st1[0, :DP] / N
    v1 = st1[1, :DP] / N - m1 * m1
    g1pad = jnp.pad(bn1_g, (0, DP - 24))
    sc1 = (lax.rsqrt(v1 + EPS) * g1pad)[None, :]
    sh1 = (jnp.pad(bn1_b, (0, DP - 24)) - m1 * sc1[0])[None, :]

    mr = m0[112:]
    vr = v0[112:]
    scr = (lax.rsqrt(vr + EPS) * normres_g)[None, :]
    shr = (normres_b - mr * scr[0])[None, :]

    w2f = jnp.pad(fc2_W, ((0, DP - 24), (0, 0)))

    h2, st2 = pl.pallas_call(
        _mix_body, grid=GRID,
        in_specs=[_rows(DP), _rows(DP), _rows1(), _rows(16), _full((1, DP)),
                  _full((1, DP)), _full((1, DP)), _full((DP, 24)),
                  _full((1, 24)), _full((1, 16)), _full((1, 16)),
                  _full((16, 24)), _full((1, 24))],
        out_specs=(_rows(48), _full((8, 128))),
        out_shape=(jax.ShapeDtypeStruct((N, 48), f32),
                   jax.ShapeDtypeStruct((8, 128), f32)),
    )(p2[0], p2[1], sin, n_res, b2p, sc1, sh1, w2f,
      fc2_b[None, :], scr, shr, fcres_W, fcres_b[None, :])
    m2 = st2[0, :48] / N
    v2 = st2[1, :48] / N - m2 * m2
    sc2 = (lax.rsqrt(v2 + EPS) * bn2_g)[None, :]
    sh2 = (bn2_b - m2 * sc2[0])[None, :]

    action = pl.pallas_call(
        _head_body, grid=GRID,
        in_specs=[_rows(48), _full((1, 48)), _full((1, 48)),
                  _full((48, 8)), _full((1, 8))],
        out_specs=_rows(8),
        out_shape=jax.ShapeDtypeStruct((N, 8), f32),
    )(h2, sc2, sh2, fc3_W, fc3_b[None, :])
    return action


# speedup vs baseline: 21.5587x; 1.0865x over previous
import functools

import jax
import jax.numpy as jnp
from jax import lax
from jax.experimental import pallas as pl
from jax.experimental.pallas import tpu as pltpu
from jax.experimental.pallas import tpu_sc as plsc

N = 50000
E = 1600000
ER = E // 128
DP = 32
NPAD = 50176
RT = NPAD // 16
LPT = 128
NB2 = 2
NSL = 3
T = 196
TLAST = 174
EPS = 1e-5

BR = 2048
GRID = (N + BR - 1) // BR


@functools.lru_cache(maxsize=None)
def _make_conv_sc():
    mesh = plsc.VectorSubcoreMesh(core_axis_name="c", subcore_axis_name="s")
    return functools.partial(
        pl.kernel,
        out_type=(jax.ShapeDtypeStruct((NPAD, DP), jnp.float32),
                  jax.ShapeDtypeStruct((NPAD, DP), jnp.float32)),
        mesh=mesh,
        scratch_types=[
            pltpu.VMEM((NSL, NB2, LPT), jnp.int32),
            pltpu.VMEM((NSL, NB2, LPT), jnp.int32),
            pltpu.VMEM((NSL, NB2, LPT, DP), jnp.float32),
            pltpu.VMEM_SHARED((NPAD, DP), jnp.float32),
            pltpu.SemaphoreType.DMA,
            pltpu.SemaphoreType.DMA,
            pltpu.SemaphoreType.DMA,
        ],
        compiler_params=pltpu.CompilerParams(use_tc_tiling_on_sc=False),
    )(_conv_sc_body)


def _conv_sc_body(h_hbm, edge_hbm, zrow_hbm, out0_hbm, out1_hbm,
                  sidx, didx, rows, agg, gsem, ssem, isem):
    cid = lax.axis_index("c")
    sid = lax.axis_index("s")
    w = sid * 2 + cid
    rbase = w * (T * NB2)
    nb = jnp.where(w == 31, TLAST, T)

    pltpu.sync_copy(zrow_hbm, agg.at[pl.ds(sid * RT, RT)])
    plsc.subcore_barrier()

    pltpu.async_copy(edge_hbm.at[0, pl.ds(rbase, NB2)], sidx.at[0], isem)
    pltpu.async_copy(edge_hbm.at[1, pl.ds(rbase, NB2)], didx.at[0], isem)

    def _drain_scatters(s):
        for i in range(NB2):
            pltpu.make_async_copy(rows.at[s, i], agg.at[didx.at[s, i]],
                                  ssem).wait()

    def _drain_gathers(s):
        for i in range(NB2):
            pltpu.make_async_copy(h_hbm.at[sidx.at[s, i]], rows.at[s, i],
                                  gsem).wait()

    def body(b, _):
        slot = lax.rem(b, NSL)

        @pl.when(b >= 2)
        def _():
            _drain_scatters(lax.rem(b + NSL - 2, NSL))

        @pl.when(b + 1 < nb)
        def _():
            ns = lax.rem(b + 1, NSL)
            r = rbase + (b + 1) * NB2
            pltpu.async_copy(edge_hbm.at[0, pl.ds(r, NB2)], sidx.at[ns], isem)
            pltpu.async_copy(edge_hbm.at[1, pl.ds(r, NB2)], didx.at[ns], isem)

        pltpu.make_async_copy(edge_hbm.at[0, pl.ds(rbase, NB2)],
                              sidx.at[slot], isem).wait()
        pltpu.make_async_copy(edge_hbm.at[1, pl.ds(rbase, NB2)],
                              didx.at[slot], isem).wait()

        for i in range(NB2):
            pltpu.async_copy(h_hbm.at[sidx.at[slot, i]], rows.at[slot, i],
                             gsem)

        @pl.when(b >= 1)
        def _():
            os_ = lax.rem(b + NSL - 1, NSL)
            _drain_gathers(os_)
            for i in range(NB2):
                pltpu.async_copy(rows.at[os_, i], agg.at[didx.at[os_, i]],
                                 ssem, add=True)
        return 0

    lax.fori_loop(0, nb, body, 0)

    ls = lax.rem(nb + NSL - 1, NSL)
    _drain_gathers(ls)
    for i in range(NB2):
        pltpu.async_copy(rows.at[ls, i], agg.at[didx.at[ls, i]], ssem,
                         add=True)
    _drain_scatters(lax.rem(nb + NSL - 2, NSL))
    _drain_scatters(ls)

    plsc.subcore_barrier()

    @pl.when(cid == 0)
    def _():
        pltpu.sync_copy(agg.at[pl.ds(sid * RT, RT)],
                        out0_hbm.at[pl.ds(sid * RT, RT)])

    @pl.when(cid == 1)
    def _():
        pltpu.sync_copy(agg.at[pl.ds(sid * RT, RT)],
                        out1_hbm.at[pl.ds(sid * RT, RT)])


@functools.lru_cache(maxsize=None)
def _make_deg_sc():
    mesh = plsc.VectorSubcoreMesh(core_axis_name="c", subcore_axis_name="s")
    return functools.partial(
        pl.kernel,
        out_type=jax.ShapeDtypeStruct((2, NPAD), jnp.float32),
        mesh=mesh,
        scratch_types=[
            pltpu.VMEM((NSL, NB2, LPT), jnp.int32),
            pltpu.VMEM((LPT,), jnp.float32),
            pltpu.VMEM_SHARED((NPAD,), jnp.float32),
            pltpu.SemaphoreType.DMA,
            pltpu.SemaphoreType.DMA,
        ],
        compiler_params=pltpu.CompilerParams(use_tc_tiling_on_sc=False),
    )(_deg_sc_body)


def _deg_sc_body(edge_hbm, zdeg_hbm, out_hbm, sidx, ones_v, counts,
                 ssem, isem):
    cid = lax.axis_index("c")
    sid = lax.axis_index("s")
    w = sid * 2 + cid
    rbase = w * (T * NB2)
    nb = jnp.where(w == 31, TLAST, T)
    for i in range(LPT // 16):
        ones_v[pl.ds(i * 16, 16)] = jnp.ones((16,), jnp.float32)
    pltpu.sync_copy(zdeg_hbm, counts.at[pl.ds(sid * RT, RT)])
    plsc.subcore_barrier()

    pltpu.async_copy(edge_hbm.at[0, pl.ds(rbase, NB2)], sidx.at[0], isem)

    def _drain(s):
        for i in range(NB2):
            pltpu.make_async_copy(ones_v, counts.at[sidx.at[s, i]],
                                  ssem).wait()

    def body(b, _):
        slot = lax.rem(b, NSL)

        @pl.when(b >= 2)
        def _():
            _drain(lax.rem(b + NSL - 2, NSL))

        @pl.when(b + 1 < nb)
        def _():
            ns = lax.rem(b + 1, NSL)
            pltpu.async_copy(edge_hbm.at[0, pl.ds(rbase + (b + 1) * NB2, NB2)],
                             sidx.at[ns], isem)

        pltpu.make_async_copy(edge_hbm.at[0, pl.ds(rbase, NB2)],
                              sidx.at[slot], isem).wait()

        for i in range(NB2):
            pltpu.async_copy(ones_v, counts.at[sidx.at[slot, i]], ssem,
                             add=True)
        return 0

    lax.fori_loop(0, nb, body, 0)
    _drain(lax.rem(nb + NSL - 2, NSL))
    _drain(lax.rem(nb + NSL - 1, NSL))
    plsc.subcore_barrier()
    pltpu.sync_copy(counts.at[pl.ds(sid * RT, RT)],
                    out_hbm.at[cid, pl.ds(sid * RT, RT)])


def _row_mask(i, br):
    rid = i * BR + lax.broadcasted_iota(jnp.int32, (BR, 1), 0)
    return rid < N


def _stats0_body(nd_ref, nr_ref, out_ref):
    i = pl.program_id(0)

    @pl.when(i == 0)
    def _():
        out_ref[...] = jnp.zeros_like(out_ref)

    m = _row_mask(i, BR)
    x = jnp.concatenate([nd_ref[...], nr_ref[...]], axis=1)
    x = jnp.where(m, x, 0.0)
    out_ref[0:1, :] += jnp.sum(x, axis=0)[None, :]
    out_ref[1:2, :] += jnp.sum(x * x, axis=0)[None, :]


def _front_body(nd_ref, nr_ref, w1d_ref, w1r_ref, b1_ref, wp_ref, dg_ref,
                out_ref):
    h1 = jax.nn.relu(
        jnp.dot(nd_ref[...], w1d_ref[...], preferred_element_type=jnp.float32)
        + jnp.dot(nr_ref[...], w1r_ref[...],
                  preferred_element_type=jnp.float32)
        + b1_ref[0:1, :])
    hw = jnp.dot(h1, wp_ref[...], preferred_element_type=jnp.float32)
    s_out = lax.rsqrt(jnp.maximum(dg_ref[...], 1.0))[:, None]
    one24 = jnp.where(
        lax.broadcasted_iota(jnp.int32, (1, DP), 1) == 24, 1.0, 0.0)
    out_ref[...] = hw * s_out + one24


def _mid_body(p0_ref, p1_ref, b1p_ref, w2_ref, dg_ref, out_ref, sin_ref):
    s = p0_ref[...] + p1_ref[...]
    ind = jnp.maximum(jnp.sum(s * jnp.where(
        lax.broadcasted_iota(jnp.int32, (1, DP), 1) == 24, 1.0, 0.0),
        axis=1, keepdims=True), 1.0)
    sin = lax.rsqrt(ind)
    g1 = s * sin + b1p_ref[0:1, :]
    hw = jnp.dot(g1, w2_ref[...], preferred_element_type=jnp.float32)
    s_out = lax.rsqrt(jnp.maximum(dg_ref[...], 1.0))[:, None]
    out_ref[...] = hw * s_out
    sin_ref[...] = sin[:, 0]


def _stats1_body(p0_ref, p1_ref, sin_ref, b2p_ref, out_ref):
    i = pl.program_id(0)

    @pl.when(i == 0)
    def _():
        out_ref[...] = jnp.zeros_like(out_ref)

    g2 = ((p0_ref[...] + p1_ref[...]) * sin_ref[...][:, None]
          + b2p_ref[0:1, :])
    g2 = jnp.where(_row_mask(i, BR), g2, 0.0)
    out_ref[0:1, :] += jnp.pad(jnp.sum(g2, axis=0), (0, 128 - DP))[None, :]
    out_ref[1:2, :] += jnp.pad(jnp.sum(g2 * g2, axis=0),
                               (0, 128 - DP))[None, :]


def _mix_body(p0_ref, p1_ref, sin_ref, nr_ref, b2p_ref, sc1_ref, sh1_ref,
              w2f_ref, b2f_ref, scr_ref, shr_ref, wr_ref, br_ref,
              out_ref, st_ref):
    i = pl.program_id(0)

    @pl.when(i == 0)
    def _():
        st_ref[...] = jnp.zeros_like(st_ref)

    g2 = ((p0_ref[...] + p1_ref[...]) * sin_ref[...][:, None]
          + b2p_ref[0:1, :])
    g2n = g2 * sc1_ref[0:1, :] + sh1_ref[0:1, :]
    ht = jax.nn.relu(
        jnp.dot(g2n, w2f_ref[...], preferred_element_type=jnp.float32)
        + b2f_ref[0:1, :])
    nrn = nr_ref[...] * scr_ref[0:1, :] + shr_ref[0:1, :]
    hres = (jnp.dot(nrn, wr_ref[...], preferred_element_type=jnp.float32)
            + br_ref[0:1, :])
    h2 = jnp.concatenate([ht, hres], axis=1)
    out_ref[...] = h2
    h2m = jnp.where(_row_mask(i, BR), h2, 0.0)
    st_ref[0:1, :] += jnp.pad(jnp.sum(h2m, axis=0), (0, 80))[None, :]
    st_ref[1:2, :] += jnp.pad(jnp.sum(h2m * h2m, axis=0), (0, 80))[None, :]


def _head_body(h2_ref, sc2_ref, sh2_ref, w3_ref, b3_ref, out_ref):
    z = (jnp.dot(h2_ref[...] * sc2_ref[0:1, :] + sh2_ref[0:1, :],
                 w3_ref[...], preferred_element_type=jnp.float32)
         + b3_ref[0:1, :])
    z = z - jnp.max(z, axis=1, keepdims=True)
    e = jnp.exp(z)
    out_ref[...] = e / jnp.sum(e, axis=1, keepdims=True)


def _full(shape):
    return pl.BlockSpec(shape, lambda i: tuple(0 for _ in shape))


def _rows(width):
    return pl.BlockSpec((BR, width), lambda i: (i, 0))


def _rows1():
    return pl.BlockSpec((BR,), lambda i: (i,))


def kernel(n_delay, n_res, edge_index, norm0_g, norm0_b, fc1_W, fc1_b,
           conv1_W, conv1_b, conv2_W, conv2_b, bn1_g, bn1_b, normres_g,
           normres_b, fcres_W, fcres_b, fc2_W, fc2_b, bn2_g, bn2_b,
           fc3_W, fc3_b):
    f32 = jnp.float32
    edge3d = edge_index.reshape(2, ER, LPT)
    zrow = jnp.zeros((RT, DP), f32)
    zdeg = jnp.zeros((RT,), f32)

    dgp = _make_deg_sc()(edge3d, zdeg)
    deg = dgp[0, :N] + dgp[1, :N]

    st0 = pl.pallas_call(
        _stats0_body, grid=GRID,
        in_specs=[_rows(112), _rows(16)],
        out_specs=_full((8, 128)),
        out_shape=jax.ShapeDtypeStruct((8, 128), f32),
    )(n_delay, n_res)
    m0 = st0[0] / N
    v0 = st0[1] / N - m0 * m0
    sc0 = lax.rsqrt(v0 + EPS) * norm0_g
    sh0 = norm0_b - m0 * sc0

    w1 = sc0[:, None] * fc1_W
    b1 = (sh0 @ fc1_W + fc1_b)[None, :]
    wp = jnp.pad(conv1_W, ((0, 0), (0, DP - 24)))
    b1p = jnp.pad(conv1_b, (0, DP - 24))[None, :]
    w2 = jnp.pad(conv2_W, ((0, DP - 24), (0, DP - 24)))
    b2p = jnp.pad(conv2_b, (0, DP - 24))[None, :]

    h1t = pl.pallas_call(
        _front_body, grid=GRID,
        in_specs=[_rows(112), _rows(16), _full((112, 40)), _full((16, 40)),
                  _full((1, 40)), _full((40, DP)), _rows1()],
        out_specs=_rows(DP),
        out_shape=jax.ShapeDtypeStruct((N, DP), f32),
    )(n_delay, n_res, w1[:112], w1[112:], b1, wp, deg)

    p1a, p1b = _make_conv_sc()(h1t, edge3d, zrow)

    h2t, sin = pl.pallas_call(
        _mid_body, grid=GRID,
        in_specs=[_rows(DP), _rows(DP), _full((1, DP)), _full((DP, DP)),
                  _rows1()],
        out_specs=(_rows(DP), _rows1()),
        out_shape=(jax.ShapeDtypeStruct((N, DP), f32),
                   jax.ShapeDtypeStruct((N,), f32)),
    )(p1a, p1b, b1p, w2, deg)

    p2a, p2b = _make_conv_sc()(h2t, edge3d, zrow)

    st1 = pl.pallas_call(
        _stats1_body, grid=GRID,
        in_specs=[_rows(DP), _rows(DP), _rows1(), _full((1, DP))],
        out_specs=_full((8, 128)),
        out_shape=jax.ShapeDtypeStruct((8, 128), f32),
    )(p2a, p2b, sin, b2p)
    m1 = st1[0, :DP] / N
    v1 = st1[1, :DP] / N - m1 * m1
    g1pad = jnp.pad(bn1_g, (0, DP - 24))
    sc1 = (lax.rsqrt(v1 + EPS) * g1pad)[None, :]
    sh1 = (jnp.pad(bn1_b, (0, DP - 24)) - m1 * sc1[0])[None, :]

    mr = m0[112:]
    vr = v0[112:]
    scr = (lax.rsqrt(vr + EPS) * normres_g)[None, :]
    shr = (normres_b - mr * scr[0])[None, :]

    w2f = jnp.pad(fc2_W, ((0, DP - 24), (0, 0)))

    h2, st2 = pl.pallas_call(
        _mix_body, grid=GRID,
        in_specs=[_rows(DP), _rows(DP), _rows1(), _rows(16), _full((1, DP)),
                  _full((1, DP)), _full((1, DP)), _full((DP, 24)),
                  _full((1, 24)), _full((1, 16)), _full((1, 16)),
                  _full((16, 24)), _full((1, 24))],
        out_specs=(_rows(48), _full((8, 128))),
        out_shape=(jax.ShapeDtypeStruct((N, 48), f32),
                   jax.ShapeDtypeStruct((8, 128), f32)),
    )(p2a, p2b, sin, n_res, b2p, sc1, sh1, w2f,
      fc2_b[None, :], scr, shr, fcres_W, fcres_b[None, :])
    m2 = st2[0, :48] / N
    v2 = st2[1, :48] / N - m2 * m2
    sc2 = (lax.rsqrt(v2 + EPS) * bn2_g)[None, :]
    sh2 = (bn2_b - m2 * sc2[0])[None, :]

    action = pl.pallas_call(
        _head_body, grid=GRID,
        in_specs=[_rows(48), _full((1, 48)), _full((1, 48)),
                  _full((48, 8)), _full((1, 8))],
        out_specs=_rows(8),
        out_shape=jax.ShapeDtypeStruct((N, 8), f32),
    )(h2, sc2, sh2, fc3_W, fc3_b[None, :])
    return action


# speedup vs baseline: 22.5047x; 1.0439x over previous
import functools

import jax
import jax.numpy as jnp
from jax import lax
from jax.experimental import pallas as pl
from jax.experimental.pallas import tpu as pltpu
from jax.experimental.pallas import tpu_sc as plsc

N = 50000
E = 1600000
ER = E // 128
DP = 32
NPAD = 50176
RT = NPAD // 16
LPT = 128
NB2 = 2
NSL = 3
T = 196
TLAST = 174
EPS = 1e-5

BR = 4096
GRID = (N + BR - 1) // BR


@functools.lru_cache(maxsize=None)
def _make_conv_sc():
    mesh = plsc.VectorSubcoreMesh(core_axis_name="c", subcore_axis_name="s")
    return functools.partial(
        pl.kernel,
        out_type=(jax.ShapeDtypeStruct((NPAD, DP), jnp.float32),
                  jax.ShapeDtypeStruct((NPAD, DP), jnp.float32)),
        mesh=mesh,
        scratch_types=[
            pltpu.VMEM((NSL, NB2, LPT), jnp.int32),
            pltpu.VMEM((NSL, NB2, LPT), jnp.int32),
            pltpu.VMEM((NSL, NB2, LPT, DP), jnp.float32),
            pltpu.VMEM_SHARED((NPAD, DP), jnp.float32),
            pltpu.SemaphoreType.DMA,
            pltpu.SemaphoreType.DMA,
            pltpu.SemaphoreType.DMA,
        ],
        compiler_params=pltpu.CompilerParams(use_tc_tiling_on_sc=False),
    )(_conv_sc_body)


def _conv_sc_body(h_hbm, edge_hbm, zrow_hbm, out0_hbm, out1_hbm,
                  sidx, didx, rows, agg, gsem, ssem, isem):
    cid = lax.axis_index("c")
    sid = lax.axis_index("s")
    w = sid * 2 + cid
    rbase = w * (T * NB2)
    nb = jnp.where(w == 31, TLAST, T)

    pltpu.sync_copy(zrow_hbm, agg.at[pl.ds(sid * RT, RT)])
    plsc.subcore_barrier()

    pltpu.async_copy(edge_hbm.at[0, pl.ds(rbase, NB2)], sidx.at[0], isem)
    pltpu.async_copy(edge_hbm.at[1, pl.ds(rbase, NB2)], didx.at[0], isem)

    def _drain_scatters(s):
        for i in range(NB2):
            pltpu.make_async_copy(rows.at[s, i], agg.at[didx.at[s, i]],
                                  ssem).wait()

    def _drain_gathers(s):
        for i in range(NB2):
            pltpu.make_async_copy(h_hbm.at[sidx.at[s, i]], rows.at[s, i],
                                  gsem).wait()

    def body(b, _):
        slot = lax.rem(b, NSL)

        @pl.when(b >= 2)
        def _():
            _drain_scatters(lax.rem(b + NSL - 2, NSL))

        @pl.when(b + 1 < nb)
        def _():
            ns = lax.rem(b + 1, NSL)
            r = rbase + (b + 1) * NB2
            pltpu.async_copy(edge_hbm.at[0, pl.ds(r, NB2)], sidx.at[ns], isem)
            pltpu.async_copy(edge_hbm.at[1, pl.ds(r, NB2)], didx.at[ns], isem)

        pltpu.make_async_copy(edge_hbm.at[0, pl.ds(rbase, NB2)],
                              sidx.at[slot], isem).wait()
        pltpu.make_async_copy(edge_hbm.at[1, pl.ds(rbase, NB2)],
                              didx.at[slot], isem).wait()

        for i in range(NB2):
            pltpu.async_copy(h_hbm.at[sidx.at[slot, i]], rows.at[slot, i],
                             gsem)

        @pl.when(b >= 1)
        def _():
            os_ = lax.rem(b + NSL - 1, NSL)
            _drain_gathers(os_)
            for i in range(NB2):
                pltpu.async_copy(rows.at[os_, i], agg.at[didx.at[os_, i]],
                                 ssem, add=True)
        return 0

    lax.fori_loop(0, nb, body, 0)

    ls = lax.rem(nb + NSL - 1, NSL)
    _drain_gathers(ls)
    for i in range(NB2):
        pltpu.async_copy(rows.at[ls, i], agg.at[didx.at[ls, i]], ssem,
                         add=True)
    _drain_scatters(lax.rem(nb + NSL - 2, NSL))
    _drain_scatters(ls)

    plsc.subcore_barrier()

    @pl.when(cid == 0)
    def _():
        pltpu.sync_copy(agg.at[pl.ds(sid * RT, RT)],
                        out0_hbm.at[pl.ds(sid * RT, RT)])

    @pl.when(cid == 1)
    def _():
        pltpu.sync_copy(agg.at[pl.ds(sid * RT, RT)],
                        out1_hbm.at[pl.ds(sid * RT, RT)])


@functools.lru_cache(maxsize=None)
def _make_deg_sc():
    mesh = plsc.VectorSubcoreMesh(core_axis_name="c", subcore_axis_name="s")
    return functools.partial(
        pl.kernel,
        out_type=jax.ShapeDtypeStruct((2, NPAD), jnp.float32),
        mesh=mesh,
        scratch_types=[
            pltpu.VMEM((NSL, NB2, LPT), jnp.int32),
            pltpu.VMEM((LPT,), jnp.float32),
            pltpu.VMEM_SHARED((NPAD,), jnp.float32),
            pltpu.SemaphoreType.DMA,
            pltpu.SemaphoreType.DMA,
        ],
        compiler_params=pltpu.CompilerParams(use_tc_tiling_on_sc=False),
    )(_deg_sc_body)


def _deg_sc_body(edge_hbm, zdeg_hbm, out_hbm, sidx, ones_v, counts,
                 ssem, isem):
    cid = lax.axis_index("c")
    sid = lax.axis_index("s")
    w = sid * 2 + cid
    rbase = w * (T * NB2)
    nb = jnp.where(w == 31, TLAST, T)
    for i in range(LPT // 16):
        ones_v[pl.ds(i * 16, 16)] = jnp.ones((16,), jnp.float32)
    pltpu.sync_copy(zdeg_hbm, counts.at[pl.ds(sid * RT, RT)])
    plsc.subcore_barrier()

    pltpu.async_copy(edge_hbm.at[0, pl.ds(rbase, NB2)], sidx.at[0], isem)

    def _drain(s):
        for i in range(NB2):
            pltpu.make_async_copy(ones_v, counts.at[sidx.at[s, i]],
                                  ssem).wait()

    def body(b, _):
        slot = lax.rem(b, NSL)

        @pl.when(b >= 2)
        def _():
            _drain(lax.rem(b + NSL - 2, NSL))

        @pl.when(b + 1 < nb)
        def _():
            ns = lax.rem(b + 1, NSL)
            pltpu.async_copy(edge_hbm.at[0, pl.ds(rbase + (b + 1) * NB2, NB2)],
                             sidx.at[ns], isem)

        pltpu.make_async_copy(edge_hbm.at[0, pl.ds(rbase, NB2)],
                              sidx.at[slot], isem).wait()

        for i in range(NB2):
            pltpu.async_copy(ones_v, counts.at[sidx.at[slot, i]], ssem,
                             add=True)
        return 0

    lax.fori_loop(0, nb, body, 0)
    _drain(lax.rem(nb + NSL - 2, NSL))
    _drain(lax.rem(nb + NSL - 1, NSL))
    plsc.subcore_barrier()
    pltpu.sync_copy(counts.at[pl.ds(sid * RT, RT)],
                    out_hbm.at[cid, pl.ds(sid * RT, RT)])


def _row_mask(i, br):
    rid = i * BR + lax.broadcasted_iota(jnp.int32, (BR, 1), 0)
    return rid < N


def _stats0_body(nd_ref, nr_ref, out_ref):
    i = pl.program_id(0)

    @pl.when(i == 0)
    def _():
        out_ref[...] = jnp.zeros_like(out_ref)

    m = _row_mask(i, BR)
    x = jnp.concatenate([nd_ref[...], nr_ref[...]], axis=1)
    x = jnp.where(m, x, 0.0)
    out_ref[0:1, :] += jnp.sum(x, axis=0)[None, :]
    out_ref[1:2, :] += jnp.sum(x * x, axis=0)[None, :]


def _front_body(nd_ref, nr_ref, w1d_ref, w1r_ref, b1_ref, wp_ref, dg_ref,
                out_ref):
    h1 = jax.nn.relu(
        jnp.dot(nd_ref[...], w1d_ref[...], preferred_element_type=jnp.float32)
        + jnp.dot(nr_ref[...], w1r_ref[...],
                  preferred_element_type=jnp.float32)
        + b1_ref[0:1, :])
    hw = jnp.dot(h1, wp_ref[...], preferred_element_type=jnp.float32)
    s_out = lax.rsqrt(jnp.maximum(dg_ref[...], 1.0))[:, None]
    one24 = jnp.where(
        lax.broadcasted_iota(jnp.int32, (1, DP), 1) == 24, 1.0, 0.0)
    out_ref[...] = hw * s_out + one24


def _mid_body(p0_ref, p1_ref, b1p_ref, w2_ref, dg_ref, out_ref, sin_ref):
    s = p0_ref[...] + p1_ref[...]
    ind = jnp.maximum(jnp.sum(s * jnp.where(
        lax.broadcasted_iota(jnp.int32, (1, DP), 1) == 24, 1.0, 0.0),
        axis=1, keepdims=True), 1.0)
    sin = lax.rsqrt(ind)
    g1 = s * sin + b1p_ref[0:1, :]
    hw = jnp.dot(g1, w2_ref[...], preferred_element_type=jnp.float32)
    s_out = lax.rsqrt(jnp.maximum(dg_ref[...], 1.0))[:, None]
    out_ref[...] = hw * s_out
    sin_ref[...] = sin[:, 0]


def _stats1_body(p0_ref, p1_ref, sin_ref, b2p_ref, out_ref):
    i = pl.program_id(0)

    @pl.when(i == 0)
    def _():
        out_ref[...] = jnp.zeros_like(out_ref)

    g2 = ((p0_ref[...] + p1_ref[...]) * sin_ref[...][:, None]
          + b2p_ref[0:1, :])
    g2 = jnp.where(_row_mask(i, BR), g2, 0.0)
    out_ref[0:1, :] += jnp.pad(jnp.sum(g2, axis=0), (0, 128 - DP))[None, :]
    out_ref[1:2, :] += jnp.pad(jnp.sum(g2 * g2, axis=0),
                               (0, 128 - DP))[None, :]


def _mix_body(p0_ref, p1_ref, sin_ref, nr_ref, b2p_ref, sc1_ref, sh1_ref,
              w2f_ref, b2f_ref, scr_ref, shr_ref, wr_ref, br_ref,
              out_ref, st_ref):
    i = pl.program_id(0)

    @pl.when(i == 0)
    def _():
        st_ref[...] = jnp.zeros_like(st_ref)

    g2 = ((p0_ref[...] + p1_ref[...]) * sin_ref[...][:, None]
          + b2p_ref[0:1, :])
    g2n = g2 * sc1_ref[0:1, :] + sh1_ref[0:1, :]
    ht = jax.nn.relu(
        jnp.dot(g2n, w2f_ref[...], preferred_element_type=jnp.float32)
        + b2f_ref[0:1, :])
    nrn = nr_ref[...] * scr_ref[0:1, :] + shr_ref[0:1, :]
    hres = (jnp.dot(nrn, wr_ref[...], preferred_element_type=jnp.float32)
            + br_ref[0:1, :])
    h2 = jnp.concatenate([ht, hres], axis=1)
    out_ref[...] = h2
    h2m = jnp.where(_row_mask(i, BR), h2, 0.0)
    st_ref[0:1, :] += jnp.pad(jnp.sum(h2m, axis=0), (0, 80))[None, :]
    st_ref[1:2, :] += jnp.pad(jnp.sum(h2m * h2m, axis=0), (0, 80))[None, :]


def _head_body(h2_ref, sc2_ref, sh2_ref, w3_ref, b3_ref, out_ref):
    z = (jnp.dot(h2_ref[...] * sc2_ref[0:1, :] + sh2_ref[0:1, :],
                 w3_ref[...], preferred_element_type=jnp.float32)
         + b3_ref[0:1, :])
    z = z - jnp.max(z, axis=1, keepdims=True)
    e = jnp.exp(z)
    out_ref[...] = e / jnp.sum(e, axis=1, keepdims=True)


def _full(shape):
    return pl.BlockSpec(shape, lambda i: tuple(0 for _ in shape))


def _rows(width):
    return pl.BlockSpec((BR, width), lambda i: (i, 0))


def _rows1():
    return pl.BlockSpec((BR,), lambda i: (i,))


def kernel(n_delay, n_res, edge_index, norm0_g, norm0_b, fc1_W, fc1_b,
           conv1_W, conv1_b, conv2_W, conv2_b, bn1_g, bn1_b, normres_g,
           normres_b, fcres_W, fcres_b, fc2_W, fc2_b, bn2_g, bn2_b,
           fc3_W, fc3_b):
    f32 = jnp.float32
    edge3d = edge_index.reshape(2, ER, LPT)
    zrow = jnp.zeros((RT, DP), f32)
    zdeg = jnp.zeros((RT,), f32)

    dgp = _make_deg_sc()(edge3d, zdeg)
    deg = dgp[0, :N] + dgp[1, :N]

    st0 = pl.pallas_call(
        _stats0_body, grid=GRID,
        in_specs=[_rows(112), _rows(16)],
        out_specs=_full((8, 128)),
        out_shape=jax.ShapeDtypeStruct((8, 128), f32),
    )(n_delay, n_res)
    m0 = st0[0] / N
    v0 = st0[1] / N - m0 * m0
    sc0 = lax.rsqrt(v0 + EPS) * norm0_g
    sh0 = norm0_b - m0 * sc0

    w1 = sc0[:, None] * fc1_W
    b1 = (sh0 @ fc1_W + fc1_b)[None, :]
    wp = jnp.pad(conv1_W, ((0, 0), (0, DP - 24)))
    b1p = jnp.pad(conv1_b, (0, DP - 24))[None, :]
    w2 = jnp.pad(conv2_W, ((0, DP - 24), (0, DP - 24)))
    b2p = jnp.pad(conv2_b, (0, DP - 24))[None, :]

    h1t = pl.pallas_call(
        _front_body, grid=GRID,
        in_specs=[_rows(112), _rows(16), _full((112, 40)), _full((16, 40)),
                  _full((1, 40)), _full((40, DP)), _rows1()],
        out_specs=_rows(DP),
        out_shape=jax.ShapeDtypeStruct((N, DP), f32),
    )(n_delay, n_res, w1[:112], w1[112:], b1, wp, deg)

    p1a, p1b = _make_conv_sc()(h1t, edge3d, zrow)

    h2t, sin = pl.pallas_call(
        _mid_body, grid=GRID,
        in_specs=[_rows(DP), _rows(DP), _full((1, DP)), _full((DP, DP)),
                  _rows1()],
        out_specs=(_rows(DP), _rows1()),
        out_shape=(jax.ShapeDtypeStruct((N, DP), f32),
                   jax.ShapeDtypeStruct((N,), f32)),
    )(p1a, p1b, b1p, w2, deg)

    p2a, p2b = _make_conv_sc()(h2t, edge3d, zrow)

    st1 = pl.pallas_call(
        _stats1_body, grid=GRID,
        in_specs=[_rows(DP), _rows(DP), _rows1(), _full((1, DP))],
        out_specs=_full((8, 128)),
        out_shape=jax.ShapeDtypeStruct((8, 128), f32),
    )(p2a, p2b, sin, b2p)
    m1 = st1[0, :DP] / N
    v1 = st1[1, :DP] / N - m1 * m1
    g1pad = jnp.pad(bn1_g, (0, DP - 24))
    sc1 = (lax.rsqrt(v1 + EPS) * g1pad)[None, :]
    sh1 = (jnp.pad(bn1_b, (0, DP - 24)) - m1 * sc1[0])[None, :]

    mr = m0[112:]
    vr = v0[112:]
    scr = (lax.rsqrt(vr + EPS) * normres_g)[None, :]
    shr = (normres_b - mr * scr[0])[None, :]

    w2f = jnp.pad(fc2_W, ((0, DP - 24), (0, 0)))

    h2, st2 = pl.pallas_call(
        _mix_body, grid=GRID,
        in_specs=[_rows(DP), _rows(DP), _rows1(), _rows(16), _full((1, DP)),
                  _full((1, DP)), _full((1, DP)), _full((DP, 24)),
                  _full((1, 24)), _full((1, 16)), _full((1, 16)),
                  _full((16, 24)), _full((1, 24))],
        out_specs=(_rows(48), _full((8, 128))),
        out_shape=(jax.ShapeDtypeStruct((N, 48), f32),
                   jax.ShapeDtypeStruct((8, 128), f32)),
    )(p2a, p2b, sin, n_res, b2p, sc1, sh1, w2f,
      fc2_b[None, :], scr, shr, fcres_W, fcres_b[None, :])
    m2 = st2[0, :48] / N
    v2 = st2[1, :48] / N - m2 * m2
    sc2 = (lax.rsqrt(v2 + EPS) * bn2_g)[None, :]
    sh2 = (bn2_b - m2 * sc2[0])[None, :]

    action = pl.pallas_call(
        _head_body, grid=GRID,
        in_specs=[_rows(48), _full((1, 48)), _full((1, 48)),
                  _full((48, 8)), _full((1, 8))],
        out_specs=_rows(8),
        out_shape=jax.ShapeDtypeStruct((N, 8), f32),
    )(h2, sc2, sh2, fc3_W, fc3_b[None, :])
    return action


# speedup vs baseline: 22.6706x; 1.0074x over previous
import functools

import jax
import jax.numpy as jnp
from jax import lax
from jax.experimental import pallas as pl
from jax.experimental.pallas import tpu as pltpu
from jax.experimental.pallas import tpu_sc as plsc

N = 50000
E = 1600000
ER = E // 128
DP = 32
NPAD = 50176
RT = NPAD // 16
LPT = 128
NB2 = 2
NSL = 3
T = 196
TLAST = 174
EPS = 1e-5

BR = 8192
GRID = (N + BR - 1) // BR


@functools.lru_cache(maxsize=None)
def _make_conv_sc():
    mesh = plsc.VectorSubcoreMesh(core_axis_name="c", subcore_axis_name="s")
    return functools.partial(
        pl.kernel,
        out_type=(jax.ShapeDtypeStruct((NPAD, DP), jnp.float32),
                  jax.ShapeDtypeStruct((NPAD, DP), jnp.float32)),
        mesh=mesh,
        scratch_types=[
            pltpu.VMEM((NSL, NB2, LPT), jnp.int32),
            pltpu.VMEM((NSL, NB2, LPT), jnp.int32),
            pltpu.VMEM((NSL, NB2, LPT, DP), jnp.float32),
            pltpu.VMEM_SHARED((NPAD, DP), jnp.float32),
            pltpu.SemaphoreType.DMA,
            pltpu.SemaphoreType.DMA,
            pltpu.SemaphoreType.DMA,
        ],
        compiler_params=pltpu.CompilerParams(use_tc_tiling_on_sc=False),
    )(_conv_sc_body)


def _conv_sc_body(h_hbm, edge_hbm, zrow_hbm, out0_hbm, out1_hbm,
                  sidx, didx, rows, agg, gsem, ssem, isem):
    cid = lax.axis_index("c")
    sid = lax.axis_index("s")
    w = sid * 2 + cid
    rbase = w * (T * NB2)
    nb = jnp.where(w == 31, TLAST, T)

    pltpu.sync_copy(zrow_hbm, agg.at[pl.ds(sid * RT, RT)])
    plsc.subcore_barrier()

    pltpu.async_copy(edge_hbm.at[0, pl.ds(rbase, NB2)], sidx.at[0], isem)
    pltpu.async_copy(edge_hbm.at[1, pl.ds(rbase, NB2)], didx.at[0], isem)

    def _drain_scatters(s):
        for i in range(NB2):
            pltpu.make_async_copy(rows.at[s, i], agg.at[didx.at[s, i]],
                                  ssem).wait()

    def _drain_gathers(s):
        for i in range(NB2):
            pltpu.make_async_copy(h_hbm.at[sidx.at[s, i]], rows.at[s, i],
                                  gsem).wait()

    def body(b, _):
        slot = lax.rem(b, NSL)

        @pl.when(b >= 2)
        def _():
            _drain_scatters(lax.rem(b + NSL - 2, NSL))

        @pl.when(b + 1 < nb)
        def _():
            ns = lax.rem(b + 1, NSL)
            r = rbase + (b + 1) * NB2
            pltpu.async_copy(edge_hbm.at[0, pl.ds(r, NB2)], sidx.at[ns], isem)
            pltpu.async_copy(edge_hbm.at[1, pl.ds(r, NB2)], didx.at[ns], isem)

        pltpu.make_async_copy(edge_hbm.at[0, pl.ds(rbase, NB2)],
                              sidx.at[slot], isem).wait()
        pltpu.make_async_copy(edge_hbm.at[1, pl.ds(rbase, NB2)],
                              didx.at[slot], isem).wait()

        for i in range(NB2):
            pltpu.async_copy(h_hbm.at[sidx.at[slot, i]], rows.at[slot, i],
                             gsem)

        @pl.when(b >= 1)
        def _():
            os_ = lax.rem(b + NSL - 1, NSL)
            _drain_gathers(os_)
            for i in range(NB2):
                pltpu.async_copy(rows.at[os_, i], agg.at[didx.at[os_, i]],
                                 ssem, add=True)
        return 0

    lax.fori_loop(0, nb, body, 0)

    ls = lax.rem(nb + NSL - 1, NSL)
    _drain_gathers(ls)
    for i in range(NB2):
        pltpu.async_copy(rows.at[ls, i], agg.at[didx.at[ls, i]], ssem,
                         add=True)
    _drain_scatters(lax.rem(nb + NSL - 2, NSL))
    _drain_scatters(ls)

    plsc.subcore_barrier()

    @pl.when(cid == 0)
    def _():
        pltpu.sync_copy(agg.at[pl.ds(sid * RT, RT)],
                        out0_hbm.at[pl.ds(sid * RT, RT)])

    @pl.when(cid == 1)
    def _():
        pltpu.sync_copy(agg.at[pl.ds(sid * RT, RT)],
                        out1_hbm.at[pl.ds(sid * RT, RT)])


@functools.lru_cache(maxsize=None)
def _make_deg_sc():
    mesh = plsc.VectorSubcoreMesh(core_axis_name="c", subcore_axis_name="s")
    return functools.partial(
        pl.kernel,
        out_type=jax.ShapeDtypeStruct((2, NPAD), jnp.float32),
        mesh=mesh,
        scratch_types=[
            pltpu.VMEM((NSL, NB2, LPT), jnp.int32),
            pltpu.VMEM((LPT,), jnp.float32),
            pltpu.VMEM_SHARED((NPAD,), jnp.float32),
            pltpu.SemaphoreType.DMA,
            pltpu.SemaphoreType.DMA,
        ],
        compiler_params=pltpu.CompilerParams(use_tc_tiling_on_sc=False),
    )(_deg_sc_body)


def _deg_sc_body(edge_hbm, zdeg_hbm, out_hbm, sidx, ones_v, counts,
                 ssem, isem):
    cid = lax.axis_index("c")
    sid = lax.axis_index("s")
    w = sid * 2 + cid
    rbase = w * (T * NB2)
    nb = jnp.where(w == 31, TLAST, T)
    for i in range(LPT // 16):
        ones_v[pl.ds(i * 16, 16)] = jnp.ones((16,), jnp.float32)
    pltpu.sync_copy(zdeg_hbm, counts.at[pl.ds(sid * RT, RT)])
    plsc.subcore_barrier()

    pltpu.async_copy(edge_hbm.at[0, pl.ds(rbase, NB2)], sidx.at[0], isem)

    def _drain(s):
        for i in range(NB2):
            pltpu.make_async_copy(ones_v, counts.at[sidx.at[s, i]],
                                  ssem).wait()

    def body(b, _):
        slot = lax.rem(b, NSL)

        @pl.when(b >= 2)
        def _():
            _drain(lax.rem(b + NSL - 2, NSL))

        @pl.when(b + 1 < nb)
        def _():
            ns = lax.rem(b + 1, NSL)
            pltpu.async_copy(edge_hbm.at[0, pl.ds(rbase + (b + 1) * NB2, NB2)],
                             sidx.at[ns], isem)

        pltpu.make_async_copy(edge_hbm.at[0, pl.ds(rbase, NB2)],
                              sidx.at[slot], isem).wait()

        for i in range(NB2):
            pltpu.async_copy(ones_v, counts.at[sidx.at[slot, i]], ssem,
                             add=True)
        return 0

    lax.fori_loop(0, nb, body, 0)
    _drain(lax.rem(nb + NSL - 2, NSL))
    _drain(lax.rem(nb + NSL - 1, NSL))
    plsc.subcore_barrier()
    pltpu.sync_copy(counts.at[pl.ds(sid * RT, RT)],
                    out_hbm.at[cid, pl.ds(sid * RT, RT)])


def _row_mask(i, br):
    rid = i * BR + lax.broadcasted_iota(jnp.int32, (BR, 1), 0)
    return rid < N


def _stats0_body(nd_ref, nr_ref, out_ref):
    i = pl.program_id(0)

    @pl.when(i == 0)
    def _():
        out_ref[...] = jnp.zeros_like(out_ref)

    m = _row_mask(i, BR)
    x = jnp.concatenate([nd_ref[...], nr_ref[...]], axis=1)
    x = jnp.where(m, x, 0.0)
    out_ref[0:1, :] += jnp.sum(x, axis=0)[None, :]
    out_ref[1:2, :] += jnp.sum(x * x, axis=0)[None, :]


def _front_body(nd_ref, nr_ref, w1d_ref, w1r_ref, b1_ref, wp_ref, dg_ref,
                out_ref):
    h1 = jax.nn.relu(
        jnp.dot(nd_ref[...], w1d_ref[...], preferred_element_type=jnp.float32)
        + jnp.dot(nr_ref[...], w1r_ref[...],
                  preferred_element_type=jnp.float32)
        + b1_ref[0:1, :])
    hw = jnp.dot(h1, wp_ref[...], preferred_element_type=jnp.float32)
    s_out = lax.rsqrt(jnp.maximum(dg_ref[...], 1.0))[:, None]
    one24 = jnp.where(
        lax.broadcasted_iota(jnp.int32, (1, DP), 1) == 24, 1.0, 0.0)
    out_ref[...] = hw * s_out + one24


def _mid_body(p0_ref, p1_ref, b1p_ref, w2_ref, dg_ref, out_ref, sin_ref):
    s = p0_ref[...] + p1_ref[...]
    ind = jnp.maximum(jnp.sum(s * jnp.where(
        lax.broadcasted_iota(jnp.int32, (1, DP), 1) == 24, 1.0, 0.0),
        axis=1, keepdims=True), 1.0)
    sin = lax.rsqrt(ind)
    g1 = s * sin + b1p_ref[0:1, :]
    hw = jnp.dot(g1, w2_ref[...], preferred_element_type=jnp.float32)
    s_out = lax.rsqrt(jnp.maximum(dg_ref[...], 1.0))[:, None]
    out_ref[...] = hw * s_out
    sin_ref[...] = sin[:, 0]


def _stats1_body(p0_ref, p1_ref, sin_ref, b2p_ref, out_ref):
    i = pl.program_id(0)

    @pl.when(i == 0)
    def _():
        out_ref[...] = jnp.zeros_like(out_ref)

    g2 = ((p0_ref[...] + p1_ref[...]) * sin_ref[...][:, None]
          + b2p_ref[0:1, :])
    g2 = jnp.where(_row_mask(i, BR), g2, 0.0)
    out_ref[0:1, :] += jnp.pad(jnp.sum(g2, axis=0), (0, 128 - DP))[None, :]
    out_ref[1:2, :] += jnp.pad(jnp.sum(g2 * g2, axis=0),
                               (0, 128 - DP))[None, :]


def _mix_body(p0_ref, p1_ref, sin_ref, nr_ref, b2p_ref, sc1_ref, sh1_ref,
              w2f_ref, b2f_ref, scr_ref, shr_ref, wr_ref, br_ref,
              out_ref, st_ref):
    i = pl.program_id(0)

    @pl.when(i == 0)
    def _():
        st_ref[...] = jnp.zeros_like(st_ref)

    g2 = ((p0_ref[...] + p1_ref[...]) * sin_ref[...][:, None]
          + b2p_ref[0:1, :])
    g2n = g2 * sc1_ref[0:1, :] + sh1_ref[0:1, :]
    ht = jax.nn.relu(
        jnp.dot(g2n, w2f_ref[...], preferred_element_type=jnp.float32)
        + b2f_ref[0:1, :])
    nrn = nr_ref[...] * scr_ref[0:1, :] + shr_ref[0:1, :]
    hres = (jnp.dot(nrn, wr_ref[...], preferred_element_type=jnp.float32)
            + br_ref[0:1, :])
    h2 = jnp.concatenate([ht, hres], axis=1)
    out_ref[...] = h2
    h2m = jnp.where(_row_mask(i, BR), h2, 0.0)
    st_ref[0:1, :] += jnp.pad(jnp.sum(h2m, axis=0), (0, 80))[None, :]
    st_ref[1:2, :] += jnp.pad(jnp.sum(h2m * h2m, axis=0), (0, 80))[None, :]


def _head_body(h2_ref, sc2_ref, sh2_ref, w3_ref, b3_ref, out_ref):
    z = (jnp.dot(h2_ref[...] * sc2_ref[0:1, :] + sh2_ref[0:1, :],
                 w3_ref[...], preferred_element_type=jnp.float32)
         + b3_ref[0:1, :])
    z = z - jnp.max(z, axis=1, keepdims=True)
    e = jnp.exp(z)
    out_ref[...] = e / jnp.sum(e, axis=1, keepdims=True)


def _full(shape):
    return pl.BlockSpec(shape, lambda i: tuple(0 for _ in shape))


def _rows(width):
    return pl.BlockSpec((BR, width), lambda i: (i, 0))


def _rows1():
    return pl.BlockSpec((BR,), lambda i: (i,))


def kernel(n_delay, n_res, edge_index, norm0_g, norm0_b, fc1_W, fc1_b,
           conv1_W, conv1_b, conv2_W, conv2_b, bn1_g, bn1_b, normres_g,
           normres_b, fcres_W, fcres_b, fc2_W, fc2_b, bn2_g, bn2_b,
           fc3_W, fc3_b):
    f32 = jnp.float32
    edge3d = edge_index.reshape(2, ER, LPT)
    zrow = jnp.zeros((RT, DP), f32)
    zdeg = jnp.zeros((RT,), f32)

    dgp = _make_deg_sc()(edge3d, zdeg)
    deg = dgp[0, :N] + dgp[1, :N]

    st0 = pl.pallas_call(
        _stats0_body, grid=GRID,
        in_specs=[_rows(112), _rows(16)],
        out_specs=_full((8, 128)),
        out_shape=jax.ShapeDtypeStruct((8, 128), f32),
    )(n_delay, n_res)
    m0 = st0[0] / N
    v0 = st0[1] / N - m0 * m0
    sc0 = lax.rsqrt(v0 + EPS) * norm0_g
    sh0 = norm0_b - m0 * sc0

    w1 = sc0[:, None] * fc1_W
    b1 = (sh0 @ fc1_W + fc1_b)[None, :]
    wp = jnp.pad(conv1_W, ((0, 0), (0, DP - 24)))
    b1p = jnp.pad(conv1_b, (0, DP - 24))[None, :]
    w2 = jnp.pad(conv2_W, ((0, DP - 24), (0, DP - 24)))
    b2p = jnp.pad(conv2_b, (0, DP - 24))[None, :]

    h1t = pl.pallas_call(
        _front_body, grid=GRID,
        in_specs=[_rows(112), _rows(16), _full((112, 40)), _full((16, 40)),
                  _full((1, 40)), _full((40, DP)), _rows1()],
        out_specs=_rows(DP),
        out_shape=jax.ShapeDtypeStruct((N, DP), f32),
    )(n_delay, n_res, w1[:112], w1[112:], b1, wp, deg)

    p1a, p1b = _make_conv_sc()(h1t, edge3d, zrow)

    h2t, sin = pl.pallas_call(
        _mid_body, grid=GRID,
        in_specs=[_rows(DP), _rows(DP), _full((1, DP)), _full((DP, DP)),
                  _rows1()],
        out_specs=(_rows(DP), _rows1()),
        out_shape=(jax.ShapeDtypeStruct((N, DP), f32),
                   jax.ShapeDtypeStruct((N,), f32)),
    )(p1a, p1b, b1p, w2, deg)

    p2a, p2b = _make_conv_sc()(h2t, edge3d, zrow)

    st1 = pl.pallas_call(
        _stats1_body, grid=GRID,
        in_specs=[_rows(DP), _rows(DP), _rows1(), _full((1, DP))],
        out_specs=_full((8, 128)),
        out_shape=jax.ShapeDtypeStruct((8, 128), f32),
    )(p2a, p2b, sin, b2p)
    m1 = st1[0, :DP] / N
    v1 = st1[1, :DP] / N - m1 * m1
    g1pad = jnp.pad(bn1_g, (0, DP - 24))
    sc1 = (lax.rsqrt(v1 + EPS) * g1pad)[None, :]
    sh1 = (jnp.pad(bn1_b, (0, DP - 24)) - m1 * sc1[0])[None, :]

    mr = m0[112:]
    vr = v0[112:]
    scr = (lax.rsqrt(vr + EPS) * normres_g)[None, :]
    shr = (normres_b - mr * scr[0])[None, :]

    w2f = jnp.pad(fc2_W, ((0, DP - 24), (0, 0)))

    h2, st2 = pl.pallas_call(
        _mix_body, grid=GRID,
        in_specs=[_rows(DP), _rows(DP), _rows1(), _rows(16), _full((1, DP)),
                  _full((1, DP)), _full((1, DP)), _full((DP, 24)),
                  _full((1, 24)), _full((1, 16)), _full((1, 16)),
                  _full((16, 24)), _full((1, 24))],
        out_specs=(_rows(48), _full((8, 128))),
        out_shape=(jax.ShapeDtypeStruct((N, 48), f32),
                   jax.ShapeDtypeStruct((8, 128), f32)),
    )(p2a, p2b, sin, n_res, b2p, sc1, sh1, w2f,
      fc2_b[None, :], scr, shr, fcres_W, fcres_b[None, :])
    m2 = st2[0, :48] / N
    v2 = st2[1, :48] / N - m2 * m2
    sc2 = (lax.rsqrt(v2 + EPS) * bn2_g)[None, :]
    sh2 = (bn2_b - m2 * sc2[0])[None, :]

    action = pl.pallas_call(
        _head_body, grid=GRID,
        in_specs=[_rows(48), _full((1, 48)), _full((1, 48)),
                  _full((48, 8)), _full((1, 8))],
        out_specs=_rows(8),
        out_shape=jax.ShapeDtypeStruct((N, 8), f32),
    )(h2, sc2, sh2, fc3_W, fc3_b[None, :])
    return action
